# Initial kernel scaffold; baseline (speedup 1.0000x reference)
import jax
import jax.numpy as jnp
from jax.experimental import pallas as pl


def kernel(x, w_gate, w1, w2, w3):
    raise NotImplementedError("write your pallas kernel here")


# speedup vs baseline: 3.2255x; 3.2255x over previous
import jax
import jax.numpy as jnp
from jax.experimental import pallas as pl
from jax.experimental.pallas import tpu as pltpu

_E = 8
_K = 2


def _moe_dense_body(x_ref, wg_ref, w1_ref, w2_ref, w3_ref, out_ref,
                    i1_ref, i2_ref, s1_ref, s2_ref):
    e = pl.program_id(0)

    @pl.when(e == 0)
    def _gate():
        x = x_ref[...]
        logits = jax.lax.dot_general(
            x, wg_ref[...], (((1,), (1,)), ((), ())),
            preferred_element_type=jnp.float32)
        cols = jax.lax.broadcasted_iota(jnp.int32, logits.shape, 1)
        m1 = jnp.max(logits, axis=1, keepdims=True)
        i1 = jnp.min(jnp.where(logits == m1, cols, _E), axis=1, keepdims=True)
        rest = jnp.where(cols == i1, -jnp.inf, logits)
        m2 = jnp.max(rest, axis=1, keepdims=True)
        i2 = jnp.min(jnp.where(rest == m2, cols, _E), axis=1, keepdims=True)
        s1 = 1.0 / (1.0 + jnp.exp(m2 - m1))
        i1_ref[...] = i1
        i2_ref[...] = i2
        s1_ref[...] = s1
        s2_ref[...] = 1.0 - s1
        out_ref[...] = jnp.zeros_like(out_ref)

    x = x_ref[...]
    w1 = w1_ref[0]
    w3 = w3_ref[0]
    w2 = w2_ref[0]
    a = jax.lax.dot_general(x, w1, (((1,), (1,)), ((), ())),
                            preferred_element_type=jnp.float32)
    b = jax.lax.dot_general(x, w3, (((1,), (1,)), ((), ())),
                            preferred_element_type=jnp.float32)
    h = a * jax.lax.logistic(a) * b
    oe = jax.lax.dot_general(h, w2, (((1,), (0,)), ((), ())),
                             preferred_element_type=jnp.float32)
    w_e = (jnp.where(i1_ref[...] == e, s1_ref[...], 0.0)
           + jnp.where(i2_ref[...] == e, s2_ref[...], 0.0))
    out_ref[...] += oe * w_e


def kernel(x, w_gate, w1, w2, w3, interpret=False):
    T, D = x.shape
    E, F, _ = w1.shape
    return pl.pallas_call(
        _moe_dense_body,
        grid=(E,),
        in_specs=[
            pl.BlockSpec((T, D), lambda e: (0, 0)),
            pl.BlockSpec((E, D), lambda e: (0, 0)),
            pl.BlockSpec((1, F, D), lambda e: (e, 0, 0)),
            pl.BlockSpec((1, F, D), lambda e: (e, 0, 0)),
            pl.BlockSpec((1, F, D), lambda e: (e, 0, 0)),
        ],
        out_specs=pl.BlockSpec((T, D), lambda e: (0, 0)),
        out_shape=jax.ShapeDtypeStruct((T, D), x.dtype),
        scratch_shapes=[pltpu.VMEM((T, 1), jnp.int32),
                        pltpu.VMEM((T, 1), jnp.int32),
                        pltpu.VMEM((T, 1), jnp.float32),
                        pltpu.VMEM((T, 1), jnp.float32)],
        compiler_params=pltpu.CompilerParams(
            dimension_semantics=("arbitrary",)),
        interpret=interpret,
    )(x, w_gate, w1, w2, w3)
